# Initial kernel scaffold; baseline (speedup 1.0000x reference)
import jax
import jax.numpy as jnp
from jax.experimental import pallas as pl


def kernel(ids, matrix):
    raise NotImplementedError("write your pallas kernel here")


# speedup vs baseline: 2.0570x; 2.0570x over previous
import functools
import math

import jax
import jax.numpy as jnp
from jax import lax
from jax.experimental import pallas as pl
from jax.experimental.pallas import tpu as pltpu
from jax.experimental.pallas import tpu_sc as plsc

DIM = 128
HALF = DIM // 2
PE_T = 50

NC = 2
NS = 16
NW = NC * NS

C = 400
SUBC = 80
NSUB = C // SUBC


def _pe_body(out_ref):
    t = lax.broadcasted_iota(jnp.int32, (PE_T, DIM), 0).astype(jnp.float32)
    d = lax.broadcasted_iota(jnp.int32, (PE_T, DIM), 1)
    dh = jnp.where(d < HALF, d, d - HALF).astype(jnp.float32)
    freq = jnp.exp(dh * (-2.0 * math.log(10000.0) / DIM))
    angle = t * freq
    out_ref[...] = jnp.where(d < HALF, jnp.sin(angle), jnp.cos(angle))


def _make_sc_kernel(n_rows):
    per_w = n_rows // NW
    n_chunks = per_w // C
    mesh = plsc.VectorSubcoreMesh(core_axis_name="c", subcore_axis_name="s")

    @functools.partial(
        pl.kernel,
        mesh=mesh,
        out_type=jax.ShapeDtypeStruct((n_rows, DIM), jnp.float32),
        scratch_types=[
            pltpu.VMEM((C,), jnp.int32),
            pltpu.VMEM((C, DIM), jnp.float32),
            pltpu.VMEM((PE_T, DIM), jnp.float32),
            pltpu.SemaphoreType.DMA,
        ],
    )
    def body(ids_hbm, pe_hbm, matrix_hbm, out_hbm, idx_v, buf, pe_v, sem):
        wid = lax.axis_index("s") * NC + lax.axis_index("c")
        base = wid * per_w
        pltpu.sync_copy(pe_hbm, pe_v)

        def chunk(ci, carry):
            cbase = base + ci * C
            pltpu.sync_copy(ids_hbm.at[pl.ds(cbase, C)], idx_v)
            handles = []
            for g in range(NSUB):
                handles.append(
                    pltpu.async_copy(
                        matrix_hbm.at[idx_v.at[pl.ds(g * SUBC, SUBC)]],
                        buf.at[pl.ds(g * SUBC, SUBC)],
                        sem,
                    )
                )
            for h in handles:
                h.wait()

            def row(r, c2):
                t = lax.rem(r, PE_T)
                for j in range(DIM // 16):
                    sl = pl.ds(j * 16, 16)
                    buf[r, sl] += pe_v[t, sl]
                return c2

            lax.fori_loop(0, C, row, 0)
            pltpu.sync_copy(buf, out_hbm.at[pl.ds(cbase, C)])
            return carry

        lax.fori_loop(0, n_chunks, chunk, 0)

    return body


def kernel(ids, matrix):
    b, hist = ids.shape
    ids_flat = (jnp.sign(ids + 1) * ids).reshape(-1)
    pe = pl.pallas_call(
        _pe_body,
        out_shape=jax.ShapeDtypeStruct((PE_T, DIM), jnp.float32),
    )()
    out = _make_sc_kernel(b * hist)(ids_flat, pe, matrix)
    return out.reshape(b, hist, DIM)


# speedup vs baseline: 3.6595x; 1.7790x over previous
import functools
import math

import jax
import jax.numpy as jnp
from jax import lax
from jax.experimental import pallas as pl
from jax.experimental.pallas import tpu as pltpu
from jax.experimental.pallas import tpu_sc as plsc

DIM = 128
HALF = DIM // 2
PE_T = 50

NC = 2
NS = 16
NW = NC * NS

C = 400
SUBC = 80
NSUB = C // SUBC


def _pe_body(out_ref):
    t = lax.broadcasted_iota(jnp.int32, (PE_T, DIM), 0).astype(jnp.float32)
    d = lax.broadcasted_iota(jnp.int32, (PE_T, DIM), 1)
    dh = jnp.where(d < HALF, d, d - HALF).astype(jnp.float32)
    freq = jnp.exp(dh * (-2.0 * math.log(10000.0) / DIM))
    angle = t * freq
    out_ref[...] = jnp.where(d < HALF, jnp.sin(angle), jnp.cos(angle))


def _make_sc_kernel(n_rows):
    per_w = n_rows // NW
    n_chunks = per_w // C
    mesh = plsc.VectorSubcoreMesh(core_axis_name="c", subcore_axis_name="s")

    @functools.partial(
        pl.kernel,
        mesh=mesh,
        out_type=jax.ShapeDtypeStruct((n_rows, DIM), jnp.float32),
        scratch_types=[
            pltpu.VMEM((C,), jnp.int32),
            pltpu.VMEM((C,), jnp.int32),
            pltpu.VMEM((C, DIM), jnp.float32),
            pltpu.VMEM((C, DIM), jnp.float32),
            pltpu.VMEM((PE_T, DIM), jnp.float32),
            pltpu.SemaphoreType.DMA,
            pltpu.SemaphoreType.DMA,
            pltpu.SemaphoreType.DMA,
            pltpu.SemaphoreType.DMA,
        ],
    )
    def body(ids_hbm, pe_hbm, matrix_hbm, out_hbm,
             idx0, idx1, buf0, buf1, pe_v, gsem0, gsem1, osem0, osem1):
        wid = lax.axis_index("s") * NC + lax.axis_index("c")
        base = wid * per_w
        pltpu.sync_copy(pe_hbm, pe_v)

        idxs = (idx0, idx1)
        bufs = (buf0, buf1)
        gsems = (gsem0, gsem1)
        osems = (osem0, osem1)

        def fire(ci, p):
            cbase = base + ci * C
            pltpu.sync_copy(ids_hbm.at[pl.ds(cbase, C)], idxs[p])
            return [
                pltpu.async_copy(
                    matrix_hbm.at[idxs[p].at[pl.ds(g * SUBC, SUBC)]],
                    bufs[p].at[pl.ds(g * SUBC, SUBC)],
                    gsems[p],
                )
                for g in range(NSUB)
            ]

        def add_pe(p):
            buf = bufs[p]

            def t_body(t, carry):
                for j in range(DIM // 16):
                    sl = pl.ds(j * 16, 16)
                    pe_reg = pe_v[t, sl]
                    for k in range(C // PE_T):
                        buf[t + PE_T * k, sl] += pe_reg
                return carry

            lax.fori_loop(0, PE_T, t_body, 0)

        gh = [None, None]
        oh = [None, None]
        gh[0] = fire(0, 0)
        for ci in range(n_chunks):
            p = ci % 2
            q = 1 - p
            if ci + 1 < n_chunks:
                if oh[q] is not None:
                    oh[q].wait()
                gh[q] = fire(ci + 1, q)
            for h in gh[p]:
                h.wait()
            add_pe(p)
            oh[p] = pltpu.async_copy(
                bufs[p], out_hbm.at[pl.ds(base + ci * C, C)], osems[p]
            )
        for h in oh:
            if h is not None:
                h.wait()

    return body


def kernel(ids, matrix):
    b, hist = ids.shape
    ids_flat = (jnp.sign(ids + 1) * ids).reshape(-1)
    pe = pl.pallas_call(
        _pe_body,
        out_shape=jax.ShapeDtypeStruct((PE_T, DIM), jnp.float32),
    )()
    out = _make_sc_kernel(b * hist)(ids_flat, pe, matrix)
    return out.reshape(b, hist, DIM)
